# baseline (speedup 1.0000x reference)
import functools

import jax
import jax.numpy as jnp
from jax import lax
from jax.experimental import pallas as pl
from jax.experimental.pallas import tpu as pltpu
from jax.experimental.pallas import tpu_sc as plsc

BATCH = 16384
DIM = 64
LANES = 16
NUM_CORES = 2
NUM_SUBCORES = 16
NUM_WORKERS = NUM_CORES * NUM_SUBCORES
B_PER_W = BATCH // NUM_WORKERS
IDX_W = 128
N_GATHER = B_PER_W // IDX_W
GROUPS = B_PER_W // LANES


def _body(uids_hbm, iids_hbm, user_hbm, item_hbm, out_hbm,
          idx_u, idx_i, rows_u, rows_i, out_v, sem):
    w = lax.axis_index("s") * NUM_CORES + lax.axis_index("c")
    base = w * B_PER_W

    pltpu.sync_copy(uids_hbm.at[pl.ds(w * N_GATHER, N_GATHER)], idx_u)
    pltpu.sync_copy(iids_hbm.at[pl.ds(w * N_GATHER, N_GATHER)], idx_i)

    copies = []
    for j in range(N_GATHER):
        copies.append(pltpu.async_copy(
            user_hbm.at[idx_u.at[j]], rows_u.at[pl.ds(j * IDX_W, IDX_W)], sem))
        copies.append(pltpu.async_copy(
            item_hbm.at[idx_i.at[j]], rows_i.at[pl.ds(j * IDX_W, IDX_W)], sem))
    for c in copies:
        c.wait()

    lane = lax.iota(jnp.int32, LANES)

    def group(g, _):
        rb = g * LANES
        row_idx = rb + lane
        acc = jnp.zeros((LANES,), jnp.float32)
        for d in range(DIM):
            col = jnp.full((LANES,), d, jnp.int32)
            u = plsc.load_gather(rows_u, [row_idx, col])
            v = plsc.load_gather(rows_i, [row_idx, col])
            acc = acc + u * v
        out_v[pl.ds(rb, LANES)] = acc
        return 0

    lax.fori_loop(0, GROUPS, group, 0)

    pltpu.sync_copy(out_v, out_hbm.at[pl.ds(base, B_PER_W)])


def kernel(user_ids, item_ids, user_table, item_table):
    uids = user_ids.astype(jnp.int32).reshape(NUM_WORKERS * N_GATHER, IDX_W)
    iids = item_ids.astype(jnp.int32).reshape(NUM_WORKERS * N_GATHER, IDX_W)

    mesh = plsc.VectorSubcoreMesh(
        core_axis_name="c", subcore_axis_name="s",
        num_cores=NUM_CORES, num_subcores=NUM_SUBCORES)

    run = pl.kernel(
        _body,
        out_type=jax.ShapeDtypeStruct((BATCH,), jnp.float32),
        mesh=mesh,
        scratch_types=[
            pltpu.VMEM((N_GATHER, IDX_W), jnp.int32),
            pltpu.VMEM((N_GATHER, IDX_W), jnp.int32),
            pltpu.VMEM((B_PER_W, DIM), jnp.float32),
            pltpu.VMEM((B_PER_W, DIM), jnp.float32),
            pltpu.VMEM((B_PER_W,), jnp.float32),
            pltpu.SemaphoreType.DMA,
        ],
        compiler_params=pltpu.CompilerParams(
            needs_layout_passes=False, use_tc_tiling_on_sc=False),
    )
    return run(uids, iids, user_table, item_table)
